# baseline (speedup 1.0000x reference)
import functools

import jax
import jax.numpy as jnp
from jax import lax
from jax.experimental import pallas as pl
from jax.experimental.pallas import tpu as pltpu
from jax.experimental.pallas import tpu_sc as plsc

_E = 16
_K = 2
_ALPHA = 0.01
_LANES = 16
_NW = 32


def _proj_kernel(x_ref, w_ref, scores_ref, ssum_ref, acc_s):
    i = pl.program_id(0)
    nb = pl.num_programs(0)

    x = x_ref[...]
    w = w_ref[...]
    logits = lax.dot_general(
        w, x, (((1,), (1,)), ((), ())),
        preferred_element_type=jnp.float32)
    m = jnp.max(logits, axis=0, keepdims=True)
    ex = jnp.exp(logits - m)
    scores = ex / jnp.sum(ex, axis=0, keepdims=True)
    scores_ref[...] = scores

    @pl.when(i == 0)
    def _():
        acc_s[...] = jnp.zeros_like(acc_s)
    acc_s[...] += jnp.sum(scores, axis=1, keepdims=True)

    @pl.when(i == nb - 1)
    def _():
        ssum_ref[...] = acc_s[...]


def _topk_sc_kernel(scores_hbm, w_hbm, i_hbm, sv, w1, w2, i1, i2, sem,
                    *, n, tpw):
    wid = lax.axis_index("c") * 16 + lax.axis_index("s")
    base = wid * tpw

    copies = [
        pltpu.make_async_copy(scores_hbm.at[pl.ds(e * n + base, tpw)],
                              sv.at[pl.ds(e * tpw, tpw)], sem)
        for e in range(_E)
    ]
    for c in copies:
        c.start()
    for c in copies:
        c.wait()

    def body(c, _):
        off = c * _LANES
        s0 = sv[pl.ds(0 * tpw + off, _LANES)]
        b1 = s0
        b2 = jnp.full((_LANES,), -1.0, jnp.float32)
        ix1 = jnp.zeros((_LANES,), jnp.int32)
        ix2 = jnp.zeros((_LANES,), jnp.int32)
        for e in range(1, _E):
            se = sv[pl.ds(e * tpw + off, _LANES)]
            ec = jnp.full((_LANES,), e, jnp.int32)
            beat1 = se > b1
            beat2 = se > b2
            b2 = jnp.where(beat1, b1, jnp.where(beat2, se, b2))
            ix2 = jnp.where(beat1, ix1, jnp.where(beat2, ec, ix2))
            b1 = jnp.where(beat1, se, b1)
            ix1 = jnp.where(beat1, ec, ix1)
        w1[pl.ds(off, _LANES)] = b1
        w2[pl.ds(off, _LANES)] = b2
        i1[pl.ds(off, _LANES)] = ix1
        i2[pl.ds(off, _LANES)] = ix2
        return _

    lax.fori_loop(0, tpw // _LANES, body, 0)

    outs = [
        pltpu.make_async_copy(w1, w_hbm.at[pl.ds(base, tpw)], sem),
        pltpu.make_async_copy(w2, w_hbm.at[pl.ds(n + base, tpw)], sem),
        pltpu.make_async_copy(i1, i_hbm.at[pl.ds(base, tpw)], sem),
        pltpu.make_async_copy(i2, i_hbm.at[pl.ds(n + base, tpw)], sem),
    ]
    for c in outs:
        c.start()
    for c in outs:
        c.wait()


def _aux_kernel(ssum_ref, idx_ref, aux_ref, *, n):
    idx = idx_ref[...]
    acc = 0.0
    for e in range(_E):
        cnt_e = jnp.sum((idx == e).astype(jnp.float32))
        acc = acc + ssum_ref[e, 0] * cnt_e
    scale = _ALPHA * _E / (float(n) * float(n) * _K)
    aux_ref[0, 0] = acc * scale


def kernel(hidden_states, weight):
    bsz, seq_len, h = hidden_states.shape
    n = bsz * seq_len
    x = hidden_states.reshape(n, h)

    block = 2048
    nb = n // block

    scores_t, ssum = pl.pallas_call(
        _proj_kernel,
        grid=(nb,),
        in_specs=[
            pl.BlockSpec((block, h), lambda i: (i, 0)),
            pl.BlockSpec((_E, h), lambda i: (0, 0)),
        ],
        out_specs=[
            pl.BlockSpec((_E, block), lambda i: (0, i)),
            pl.BlockSpec((_E, 1), lambda i: (0, 0)),
        ],
        out_shape=[
            jax.ShapeDtypeStruct((_E, n), jnp.float32),
            jax.ShapeDtypeStruct((_E, 1), jnp.float32),
        ],
        scratch_shapes=[pltpu.VMEM((_E, 1), jnp.float32)],
    )(x, weight)

    scores_lin = scores_t.reshape(-1)
    tpw = n // _NW

    mesh = plsc.VectorSubcoreMesh(core_axis_name="c", subcore_axis_name="s",
                                  num_cores=2, num_subcores=16)
    sc = pl.kernel(
        functools.partial(_topk_sc_kernel, n=n, tpw=tpw),
        mesh=mesh,
        out_type=[
            jax.ShapeDtypeStruct((_K * n,), jnp.float32),
            jax.ShapeDtypeStruct((_K * n,), jnp.int32),
        ],
        scratch_types=[
            pltpu.VMEM((_E * tpw,), jnp.float32),
            pltpu.VMEM((tpw,), jnp.float32),
            pltpu.VMEM((tpw,), jnp.float32),
            pltpu.VMEM((tpw,), jnp.int32),
            pltpu.VMEM((tpw,), jnp.int32),
            pltpu.SemaphoreType.DMA,
        ],
    )
    w_lin, i_lin = sc(scores_lin)

    idx2d = i_lin.reshape(_K * n // 128, 128)
    aux = pl.pallas_call(
        functools.partial(_aux_kernel, n=n),
        in_specs=[
            pl.BlockSpec((_E, 1), lambda: (0, 0)),
            pl.BlockSpec((_K * n // 128, 128), lambda: (0, 0)),
        ],
        out_specs=pl.BlockSpec(memory_space=pltpu.SMEM),
        out_shape=jax.ShapeDtypeStruct((1, 1), jnp.float32),
    )(ssum, idx2d)

    topk_idx = i_lin.reshape(_K, n).T
    topk_weight = w_lin.reshape(_K, n).T
    return topk_idx, topk_weight, aux[0, 0]
